# baseline (speedup 1.0000x reference)
import functools

import jax
import jax.numpy as jnp
from jax import lax
from jax.experimental import pallas as pl
from jax.experimental.pallas import tpu as pltpu
from jax.experimental.pallas import tpu_sc as plsc

SEQ_LEN = 8192
BLEN = 32
EMB = 128

_NC = 2
_NS = 16
_NW = _NC * _NS
_SEQ_W = SEQ_LEN // _NW
_RB = 8
_NBLK = _SEQ_W // _RB


def _sc_body(bin_hbm, tab_hbm, out_hbm, bin_v, tab_v, st0, st1, sem0, sem1):
    wid = lax.axis_index("s") * _NC + lax.axis_index("c")
    base = wid * _SEQ_W
    pltpu.sync_copy(bin_hbm.at[pl.ds(base, _SEQ_W)], bin_v)
    pltpu.sync_copy(tab_hbm, tab_v)

    def block(j, stage):
        def row(r, _):
            s = j * _RB + r
            amp_lo = bin_v[s, pl.ds(0, 16)] * 2.0 - 1.0
            amp_hi = bin_v[s, pl.ds(16, 16)] * 2.0 - 1.0
            for b in range(BLEN):
                sgn = amp_lo[b] if b < 16 else amp_hi[b - 16]
                for e in range(EMB // 16):
                    stage[r, b, pl.ds(e * 16, 16)] = (
                        sgn * tab_v[b, pl.ds(e * 16, 16)])
            return 0

        lax.fori_loop(0, _RB, row, 0)

    def outer(jj, _):
        for t, stage, sem in ((0, st0, sem0), (1, st1, sem1)):
            j = jj * 2 + t

            @pl.when(jj > 0)
            def _wait():
                pltpu.make_async_copy(out_hbm.at[pl.ds(0, _RB)], stage,
                                      sem).wait()

            block(j, stage)
            pltpu.make_async_copy(
                stage, out_hbm.at[pl.ds(base + j * _RB, _RB)], sem).start()
        return 0

    lax.fori_loop(0, _NBLK // 2, outer, 0)
    pltpu.make_async_copy(out_hbm.at[pl.ds(0, _RB)], st0, sem0).wait()
    pltpu.make_async_copy(out_hbm.at[pl.ds(0, _RB)], st1, sem1).wait()


def _sc_emb(binary_input, embeddings):
    mesh = plsc.VectorSubcoreMesh(core_axis_name="c", subcore_axis_name="s")
    return pl.kernel(
        _sc_body,
        out_type=jax.ShapeDtypeStruct((SEQ_LEN, BLEN, EMB), jnp.float32),
        mesh=mesh,
        scratch_types=[
            pltpu.VMEM((_SEQ_W, BLEN), jnp.float32),
            pltpu.VMEM((BLEN, EMB), jnp.float32),
            pltpu.VMEM((_RB, BLEN, EMB), jnp.float32),
            pltpu.VMEM((_RB, BLEN, EMB), jnp.float32),
            pltpu.SemaphoreType.DMA,
            pltpu.SemaphoreType.DMA,
        ],
    )(binary_input, embeddings)


def _logit_body(bin_ref, emb_ref, logit_ref):
    amp = bin_ref[...] * 2.0 - 1.0
    rowsum = jnp.sum(emb_ref[...], axis=1)
    logit_ref[...] = amp * rowsum[None, :]


def _tc_logit(binary_input, embeddings):
    return pl.pallas_call(
        _logit_body,
        out_shape=jax.ShapeDtypeStruct((SEQ_LEN, BLEN), jnp.float32),
    )(binary_input, embeddings)


@jax.jit
def _run(binary_input, embeddings):
    emb = _sc_emb(binary_input, embeddings)
    logit = _tc_logit(binary_input, embeddings)
    return emb, logit.reshape(SEQ_LEN, BLEN, 1)


def kernel(binary_input, embeddings):
    return _run(binary_input, embeddings)


# speedup vs baseline: 4.6812x; 4.6812x over previous
import jax
import jax.numpy as jnp
from jax import lax
from jax.experimental import pallas as pl
from jax.experimental.pallas import tpu as pltpu
from jax.experimental.pallas import tpu_sc as plsc

SEQ_LEN = 8192
BLEN = 32
EMB = 128

_SEQ_BLK = 512

_NC = 2
_NS = 16
_NW = _NC * _NS
_SEQ_W = SEQ_LEN // _NW


def _emb_body(bin_ref, emb_ref, out_ref):
    amp = bin_ref[...] * 2.0 - 1.0
    table = emb_ref[...]
    out_ref[...] = amp[:, :, None] * table[None, :, :]


def _tc_emb(binary_input, embeddings):
    return pl.pallas_call(
        _emb_body,
        grid=(SEQ_LEN // _SEQ_BLK,),
        in_specs=[
            pl.BlockSpec((_SEQ_BLK, BLEN), lambda i: (i, 0)),
            pl.BlockSpec((BLEN, EMB), lambda i: (0, 0)),
        ],
        out_specs=pl.BlockSpec((_SEQ_BLK, BLEN, EMB), lambda i: (i, 0, 0)),
        out_shape=jax.ShapeDtypeStruct((SEQ_LEN, BLEN, EMB), jnp.float32),
    )(binary_input, embeddings)


def _rs_body(emb_ref, rs_ref):
    rs_ref[...] = jnp.sum(emb_ref[...], axis=1, keepdims=True).T


def _tc_rowsums(embeddings):
    return pl.pallas_call(
        _rs_body,
        out_shape=jax.ShapeDtypeStruct((1, BLEN), jnp.float32),
    )(embeddings)


def _logit_body(bin_hbm, rs_hbm, logit_hbm, bin_v, rs_v, logit_v):
    wid = lax.axis_index("s") * _NC + lax.axis_index("c")
    base = wid * _SEQ_W
    pltpu.sync_copy(bin_hbm.at[pl.ds(base, _SEQ_W)], bin_v)
    pltpu.sync_copy(rs_hbm, rs_v)
    rs_lo = rs_v[0, pl.ds(0, 16)]
    rs_hi = rs_v[0, pl.ds(16, 16)]

    def row(s, _):
        for q in range(4):
            amp_lo = bin_v[s * 4 + q, pl.ds(0, 16)] * 2.0 - 1.0
            amp_hi = bin_v[s * 4 + q, pl.ds(16, 16)] * 2.0 - 1.0
            logit_v[s * 4 + q, pl.ds(0, 16)] = amp_lo * rs_lo
            logit_v[s * 4 + q, pl.ds(16, 16)] = amp_hi * rs_hi
        return 0

    lax.fori_loop(0, _SEQ_W // 4, row, 0)
    pltpu.sync_copy(logit_v, logit_hbm.at[pl.ds(base, _SEQ_W)])


def _sc_logit(binary_input, rowsums):
    mesh = plsc.VectorSubcoreMesh(core_axis_name="c", subcore_axis_name="s")
    return pl.kernel(
        _logit_body,
        out_type=jax.ShapeDtypeStruct((SEQ_LEN, BLEN), jnp.float32),
        mesh=mesh,
        scratch_types=[
            pltpu.VMEM((_SEQ_W, BLEN), jnp.float32),
            pltpu.VMEM((1, BLEN), jnp.float32),
            pltpu.VMEM((_SEQ_W, BLEN), jnp.float32),
        ],
    )(binary_input, rowsums)


@jax.jit
def _run(binary_input, embeddings):
    rowsums = _tc_rowsums(embeddings)
    logit = _sc_logit(binary_input, rowsums)
    emb = _tc_emb(binary_input, embeddings)
    return emb, logit.reshape(SEQ_LEN, BLEN, 1)


def kernel(binary_input, embeddings):
    return _run(binary_input, embeddings)
